# baseline (device time: 44043 ns/iter reference)
import functools

import jax
import jax.numpy as jnp
from jax import lax
from jax.experimental import pallas as pl
from jax.experimental.pallas import tpu as pltpu

N_Z = 4
N_CHUNK = 4


def kernel(x):
    m_per, n = x.shape
    n_per = n // N_Z
    out_rows = N_Z * m_per
    half = m_per // 2
    chunk = half // N_CHUNK

    def body(x_ref, out_ref, zsend_sems, zrecv_sems, xsend_sems, xrecv_sems):
        my_x = lax.axis_index("x")
        my_y = lax.axis_index("y")
        my_z = lax.axis_index("z")
        row_off = my_x * half

        barrier_sem = pltpu.get_barrier_semaphore()
        for d in range(1, N_Z):
            pz = lax.rem(my_z + d, N_Z)
            pl.semaphore_signal(
                barrier_sem, inc=1,
                device_id=(my_x, my_y, pz),
                device_id_type=pl.DeviceIdType.MESH,
            )
        pl.semaphore_signal(
            barrier_sem, inc=1,
            device_id=(1 - my_x, my_y, my_z),
            device_id_type=pl.DeviceIdType.MESH,
        )
        pl.semaphore_wait(barrier_sem, N_Z)

        def make_z(c, d):
            tz = lax.rem(my_z + d, N_Z)
            s = (d - 1) * N_CHUNK + c
            return pltpu.make_async_remote_copy(
                src_ref=x_ref.at[
                    pl.ds(row_off + c * chunk, chunk),
                    pl.ds(tz * n_per, n_per),
                ],
                dst_ref=out_ref.at[
                    pl.ds(my_z * m_per + row_off + c * chunk, chunk), :
                ],
                send_sem=zsend_sems.at[s],
                recv_sem=zrecv_sems.at[s],
                device_id=(my_x, my_y, tz),
                device_id_type=pl.DeviceIdType.MESH,
            )

        def make_fwd(c, d):
            sz = lax.rem(my_z - d + N_Z, N_Z)
            s = (d - 1) * N_CHUNK + c
            rows = pl.ds(sz * m_per + row_off + c * chunk, chunk)
            return pltpu.make_async_remote_copy(
                src_ref=out_ref.at[rows, :],
                dst_ref=out_ref.at[rows, :],
                send_sem=xsend_sems.at[s],
                recv_sem=xrecv_sems.at[s],
                device_id=(1 - my_x, my_y, my_z),
                device_id_type=pl.DeviceIdType.MESH,
            )

        z_salvos = []
        x_rdmas = []
        x_prev = None
        for c in range(N_CHUNK):
            if c > 0:
                for r in z_salvos[c - 1]:
                    r.wait_send()
            salvo = [make_z(c, d) for d in range(1, N_Z)]
            for r in salvo:
                r.start()
            z_salvos.append(salvo)
            if c == 0:
                out_ref[pl.ds(my_z * m_per, m_per), :] = x_ref[
                    :, pl.ds(my_z * n_per, n_per)
                ]
            else:
                for d in range(1, N_Z):
                    z_salvos[c - 1][d - 1].wait_recv()
                    fwd = make_fwd(c - 1, d)
                    if x_prev is not None:
                        x_prev.wait_send()
                    fwd.start()
                    x_rdmas.append(fwd)
                    x_prev = fwd

        for d in range(1, N_Z):
            z_salvos[-1][d - 1].wait_recv()
            fwd = make_fwd(N_CHUNK - 1, d)
            x_prev.wait_send()
            fwd.start()
            x_rdmas.append(fwd)
            x_prev = fwd
        for r in z_salvos[-1]:
            r.wait_send()
        x_prev.wait_send()
        for fwd in x_rdmas:
            fwd.wait_recv()

        @functools.partial(
            pl.run_scoped, second_barrier=pltpu.SemaphoreType.REGULAR
        )
        def _(second_barrier):
            for d in range(1, N_Z):
                pz = lax.rem(my_z + d, N_Z)
                pl.semaphore_signal(
                    second_barrier, inc=1,
                    device_id=(my_x, my_y, pz),
                    device_id_type=pl.DeviceIdType.MESH,
                )
            pl.semaphore_signal(
                second_barrier, inc=1,
                device_id=(1 - my_x, my_y, my_z),
                device_id_type=pl.DeviceIdType.MESH,
            )
            pl.semaphore_wait(second_barrier, N_Z)

    return pl.pallas_call(
        body,
        out_shape=jax.ShapeDtypeStruct((out_rows, n_per), x.dtype),
        in_specs=[pl.BlockSpec(memory_space=pltpu.VMEM)],
        out_specs=pl.BlockSpec(memory_space=pltpu.VMEM),
        scratch_shapes=[
            pltpu.SemaphoreType.DMA(((N_Z - 1) * N_CHUNK,)),
            pltpu.SemaphoreType.DMA(((N_Z - 1) * N_CHUNK,)),
            pltpu.SemaphoreType.DMA(((N_Z - 1) * N_CHUNK,)),
            pltpu.SemaphoreType.DMA(((N_Z - 1) * N_CHUNK,)),
        ],
        compiler_params=pltpu.CompilerParams(collective_id=0),
    )(x)


# device time: 41235 ns/iter; 1.0681x vs baseline; 1.0681x over previous
import functools

import jax
import jax.numpy as jnp
from jax import lax
from jax.experimental import pallas as pl
from jax.experimental.pallas import tpu as pltpu

N_Z = 4
N_CHUNK = 4


def kernel(x):
    m_per, n = x.shape
    n_per = n // N_Z
    out_rows = N_Z * m_per
    half = m_per // 2
    chunk = half // N_CHUNK

    def body(x_ref, out_ref, zsend_sems, zrecv_sems, xsend_sems, xrecv_sems):
        my_x = lax.axis_index("x")
        my_y = lax.axis_index("y")
        my_z = lax.axis_index("z")
        row_off = my_x * half

        barrier_sem = pltpu.get_barrier_semaphore()
        for d in range(1, N_Z):
            pz = lax.rem(my_z + d, N_Z)
            pl.semaphore_signal(
                barrier_sem, inc=1,
                device_id=(my_x, my_y, pz),
                device_id_type=pl.DeviceIdType.MESH,
            )
        pl.semaphore_signal(
            barrier_sem, inc=1,
            device_id=(1 - my_x, my_y, my_z),
            device_id_type=pl.DeviceIdType.MESH,
        )
        pl.semaphore_wait(barrier_sem, N_Z)

        z_rdmas = []
        for c in range(N_CHUNK):
            for d in range(1, N_Z):
                tz = lax.rem(my_z + d, N_Z)
                s = (d - 1) * N_CHUNK + c
                rdma = pltpu.make_async_remote_copy(
                    src_ref=x_ref.at[
                        pl.ds(row_off + c * chunk, chunk),
                        pl.ds(tz * n_per, n_per),
                    ],
                    dst_ref=out_ref.at[
                        pl.ds(my_z * m_per + row_off + c * chunk, chunk), :
                    ],
                    send_sem=zsend_sems.at[s],
                    recv_sem=zrecv_sems.at[s],
                    device_id=(my_x, my_y, tz),
                    device_id_type=pl.DeviceIdType.MESH,
                )
                rdma.start()
                z_rdmas.append(rdma)

        out_ref[pl.ds(my_z * m_per, m_per), :] = x_ref[:, pl.ds(my_z * n_per, n_per)]

        x_rdmas = []
        for c in range(N_CHUNK):
            for d in range(1, N_Z):
                sz = lax.rem(my_z - d + N_Z, N_Z)
                s = (d - 1) * N_CHUNK + c
                z_rdmas[c * (N_Z - 1) + (d - 1)].wait_recv()
                fwd = pltpu.make_async_remote_copy(
                    src_ref=out_ref.at[
                        pl.ds(sz * m_per + row_off + c * chunk, chunk), :
                    ],
                    dst_ref=out_ref.at[
                        pl.ds(sz * m_per + row_off + c * chunk, chunk), :
                    ],
                    send_sem=xsend_sems.at[s],
                    recv_sem=xrecv_sems.at[s],
                    device_id=(1 - my_x, my_y, my_z),
                    device_id_type=pl.DeviceIdType.MESH,
                )
                fwd.start()
                x_rdmas.append(fwd)

        for i, fwd in enumerate(x_rdmas):
            fwd.wait_recv()
            fwd.wait_send()
            z_rdmas[i].wait_send()

        @functools.partial(
            pl.run_scoped, second_barrier=pltpu.SemaphoreType.REGULAR
        )
        def _(second_barrier):
            for d in range(1, N_Z):
                pz = lax.rem(my_z + d, N_Z)
                pl.semaphore_signal(
                    second_barrier, inc=1,
                    device_id=(my_x, my_y, pz),
                    device_id_type=pl.DeviceIdType.MESH,
                )
            pl.semaphore_signal(
                second_barrier, inc=1,
                device_id=(1 - my_x, my_y, my_z),
                device_id_type=pl.DeviceIdType.MESH,
            )
            pl.semaphore_wait(second_barrier, N_Z)

    return pl.pallas_call(
        body,
        out_shape=jax.ShapeDtypeStruct((out_rows, n_per), x.dtype),
        in_specs=[pl.BlockSpec(memory_space=pltpu.VMEM)],
        out_specs=pl.BlockSpec(memory_space=pltpu.VMEM),
        scratch_shapes=[
            pltpu.SemaphoreType.DMA(((N_Z - 1) * N_CHUNK,)),
            pltpu.SemaphoreType.DMA(((N_Z - 1) * N_CHUNK,)),
            pltpu.SemaphoreType.DMA(((N_Z - 1) * N_CHUNK,)),
            pltpu.SemaphoreType.DMA(((N_Z - 1) * N_CHUNK,)),
        ],
        compiler_params=pltpu.CompilerParams(collective_id=0),
    )(x)


# device time: 37718 ns/iter; 1.1677x vs baseline; 1.0932x over previous
import functools

import jax
import jax.numpy as jnp
from jax import lax
from jax.experimental import pallas as pl
from jax.experimental.pallas import tpu as pltpu

N_Z = 4
N_CHUNK = 4


def kernel(x):
    m_per, n = x.shape
    n_per = n // N_Z
    out_rows = N_Z * m_per
    half = m_per // 2
    chunk = half // N_CHUNK

    def body(x_ref, out_ref, zsend_sems, zrecv_sems, xsend_sems, xrecv_sems):
        my_x = lax.axis_index("x")
        my_y = lax.axis_index("y")
        my_z = lax.axis_index("z")
        row_off = my_x * half

        barrier_sem = pltpu.get_barrier_semaphore()
        for d in range(1, N_Z):
            pz = lax.rem(my_z + d, N_Z)
            pl.semaphore_signal(
                barrier_sem, inc=1,
                device_id=(my_x, my_y, pz),
                device_id_type=pl.DeviceIdType.MESH,
            )
        pl.semaphore_signal(
            barrier_sem, inc=1,
            device_id=(1 - my_x, my_y, my_z),
            device_id_type=pl.DeviceIdType.MESH,
        )
        pl.semaphore_wait(barrier_sem, N_Z)

        z_rdmas = []
        for c in range(N_CHUNK):
            for d in range(1, N_Z):
                tz = lax.rem(my_z + d, N_Z)
                s = (d - 1) * N_CHUNK + c
                rdma = pltpu.make_async_remote_copy(
                    src_ref=x_ref.at[
                        pl.ds(row_off + c * chunk, chunk),
                        pl.ds(tz * n_per, n_per),
                    ],
                    dst_ref=out_ref.at[
                        pl.ds(my_z * m_per + row_off + c * chunk, chunk), :
                    ],
                    send_sem=zsend_sems.at[s],
                    recv_sem=zrecv_sems.at[s],
                    device_id=(my_x, my_y, tz),
                    device_id_type=pl.DeviceIdType.MESH,
                )
                rdma.start()
                z_rdmas.append(rdma)

        out_ref[pl.ds(my_z * m_per, m_per), :] = x_ref[:, pl.ds(my_z * n_per, n_per)]

        x_rdmas = []
        for c in range(N_CHUNK):
            for d in range(1, N_Z):
                sz = lax.rem(my_z - d + N_Z, N_Z)
                s = (d - 1) * N_CHUNK + c
                z_rdmas[c * (N_Z - 1) + (d - 1)].wait_recv()
                fwd = pltpu.make_async_remote_copy(
                    src_ref=out_ref.at[
                        pl.ds(sz * m_per + row_off + c * chunk, chunk), :
                    ],
                    dst_ref=out_ref.at[
                        pl.ds(sz * m_per + row_off + c * chunk, chunk), :
                    ],
                    send_sem=xsend_sems.at[s],
                    recv_sem=xrecv_sems.at[s],
                    device_id=(1 - my_x, my_y, my_z),
                    device_id_type=pl.DeviceIdType.MESH,
                )
                fwd.start()
                x_rdmas.append(fwd)

        for i, fwd in enumerate(x_rdmas):
            fwd.wait_recv()
            fwd.wait_send()
            z_rdmas[i].wait_send()


    return pl.pallas_call(
        body,
        out_shape=jax.ShapeDtypeStruct((out_rows, n_per), x.dtype),
        in_specs=[pl.BlockSpec(memory_space=pltpu.VMEM)],
        out_specs=pl.BlockSpec(memory_space=pltpu.VMEM),
        scratch_shapes=[
            pltpu.SemaphoreType.DMA(((N_Z - 1) * N_CHUNK,)),
            pltpu.SemaphoreType.DMA(((N_Z - 1) * N_CHUNK,)),
            pltpu.SemaphoreType.DMA(((N_Z - 1) * N_CHUNK,)),
            pltpu.SemaphoreType.DMA(((N_Z - 1) * N_CHUNK,)),
        ],
        compiler_params=pltpu.CompilerParams(collective_id=0),
    )(x)
